# baseline (speedup 1.0000x reference)
import functools

import jax
import jax.numpy as jnp
from jax import lax
from jax.experimental import pallas as pl
from jax.experimental.pallas import tpu as pltpu


def _bb_kernel(x_ref, wa1_ref, wb1_ref, wc1_ref, wa2_ref, wb2_ref, wc2_ref,
               s_ref, b1_ref, b2_ref, out_ref, col_ref, *, H, W):
    _, C, _, _ = x_ref.shape

    x = x_ref[0]

    def conv3x3(ah, w_refs):
        col_ref[1:H, 0 * C:1 * C, :] = ah[0:H - 1]
        col_ref[0:1, 0 * C:1 * C, :] = jnp.zeros((1, C, W), jnp.float32)
        col_ref[:, 1 * C:2 * C, :] = ah
        col_ref[0:H - 1, 2 * C:3 * C, :] = ah[1:H]
        col_ref[H - 1:H, 2 * C:3 * C, :] = jnp.zeros((1, C, W), jnp.float32)
        col = col_ref[...]
        acc = None
        for kx in range(3):
            g = lax.dot_general(
                w_refs[kx][...], col,
                dimension_numbers=(((1,), (1,)), ((), ())),
                preferred_element_type=jnp.float32)
            g = lax.dot_general(
                g, s_ref[kx],
                dimension_numbers=(((2,), (0,)), ((), ())),
                preferred_element_type=jnp.float32)
            acc = g if acc is None else acc + g
        return acc

    xh = jnp.transpose(x, (1, 0, 2))
    y1 = jnp.maximum(conv3x3(xh, (wa1_ref, wb1_ref, wc1_ref)) + b1_ref[...],
                     0.0)
    y1h = jnp.transpose(y1, (1, 0, 2))
    y2 = jnp.maximum(conv3x3(y1h, (wa2_ref, wb2_ref, wc2_ref)) + b2_ref[...]
                     + x, 0.0)
    out_ref[0] = y2.astype(out_ref.dtype)


def _fold_bn(gamma, beta, mean, var, eps=1e-5):
    scale = gamma / jnp.sqrt(var + eps)
    bias = beta - mean * scale
    return scale, bias


def _prep_weight(w_oihw, scale):
    w = w_oihw.astype(jnp.float32) * scale[:, None, None, None]
    w = jnp.transpose(w, (0, 2, 1, 3))
    o, kh, i, kw = w.shape
    return [w[:, :, :, kx].reshape(o, kh * i) for kx in range(3)]


def kernel(x, w1, w2, bn1_gamma, bn1_beta, bn1_mean, bn1_var,
           bn2_gamma, bn2_beta, bn2_mean, bn2_var):
    N, C, H, W = x.shape

    s1, b1 = _fold_bn(bn1_gamma, bn1_beta, bn1_mean, bn1_var)
    s2, b2 = _fold_bn(bn2_gamma, bn2_beta, bn2_mean, bn2_var)
    wa1, wb1, wc1 = _prep_weight(w1, s1)
    wa2, wb2, wc2 = _prep_weight(w2, s2)
    b1c = b1.reshape(C, 1, 1).astype(jnp.float32)
    b2c = b2.reshape(C, 1, 1).astype(jnp.float32)

    v = jnp.arange(W)
    smats = jnp.stack([
        (v[:, None] == v[None, :] + (kx - 1)).astype(jnp.float32)
        for kx in range(3)])

    kernel_fn = functools.partial(_bb_kernel, H=H, W=W)

    out = pl.pallas_call(
        kernel_fn,
        out_shape=jax.ShapeDtypeStruct((N, C, H, W), x.dtype),
        grid=(N,),
        in_specs=[
            pl.BlockSpec((1, C, H, W), lambda b: (b, 0, 0, 0)),
            pl.BlockSpec((C, 3 * C), lambda b: (0, 0)),
            pl.BlockSpec((C, 3 * C), lambda b: (0, 0)),
            pl.BlockSpec((C, 3 * C), lambda b: (0, 0)),
            pl.BlockSpec((C, 3 * C), lambda b: (0, 0)),
            pl.BlockSpec((C, 3 * C), lambda b: (0, 0)),
            pl.BlockSpec((C, 3 * C), lambda b: (0, 0)),
            pl.BlockSpec((3, W, W), lambda b: (0, 0, 0)),
            pl.BlockSpec((C, 1, 1), lambda b: (0, 0, 0)),
            pl.BlockSpec((C, 1, 1), lambda b: (0, 0, 0)),
        ],
        out_specs=pl.BlockSpec((1, C, H, W), lambda b: (b, 0, 0, 0)),
        scratch_shapes=[
            pltpu.VMEM((H, 3 * C, W), jnp.float32),
        ],
        compiler_params=pltpu.CompilerParams(
            dimension_semantics=("parallel",),
            vmem_limit_bytes=48 * 1024 * 1024,
        ),
    )(x, wa1, wb1, wc1, wa2, wb2, wc2, smats, b1c, b2c)

    return out


# speedup vs baseline: 1.2019x; 1.2019x over previous
import functools

import jax
import jax.numpy as jnp
from jax import lax
from jax.experimental import pallas as pl
from jax.experimental.pallas import tpu as pltpu


def _bb_kernel(x_ref, w1_ref, b1_ref, w2_ref, b2_ref, out_ref,
               dense_ref, col_ref, *, H, W):
    C = x_ref.shape[0]
    L = H * W

    x3 = x_ref[...]
    for h in range(H):
        dense_ref[:, h * W:(h + 1) * W] = x3[:, h, :]
    x = dense_ref[...]

    lane = lax.broadcasted_iota(jnp.int32, (1, L), 1)
    colx = lane % W
    rowy = lane // W
    col_m = [colx >= 1, None, colx <= W - 2]
    row_m = [rowy >= 1, None, rowy <= H - 2]
    masks = []
    for ky in range(3):
        for kx in range(3):
            m = row_m[ky]
            if col_m[kx] is not None:
                m = col_m[kx] if m is None else jnp.logical_and(m, col_m[kx])
            masks.append(None if m is None else jnp.where(m, 1.0, 0.0))

    def conv3x3(a, w_ref_loc):
        for ky in range(3):
            for kx in range(3):
                tap = ky * 3 + kx
                s = (ky - 1) * W + (kx - 1)
                t = a if s == 0 else pltpu.roll(a, shift=(-s) % L, axis=1)
                m = masks[tap]
                if m is not None:
                    t = t * m
                col_ref[tap * C:(tap + 1) * C, :] = t.astype(jnp.bfloat16)
        return jnp.dot(w_ref_loc[...], col_ref[...],
                       preferred_element_type=jnp.float32)

    y1 = jnp.maximum(conv3x3(x, w1_ref) + b1_ref[...], 0.0)
    y2 = jnp.maximum(conv3x3(y1, w2_ref) + b2_ref[...] + x, 0.0)
    y2 = y2.astype(out_ref.dtype)

    for h in range(H):
        out_ref[:, h, :] = y2[:, h * W:(h + 1) * W]


def _fold_bn(gamma, beta, mean, var, eps=1e-5):
    scale = gamma / jnp.sqrt(var + eps)
    bias = beta - mean * scale
    return scale, bias


def _prep_weight(w_oihw, scale):
    w = w_oihw.astype(jnp.float32) * scale[:, None, None, None]
    w = jnp.transpose(w, (0, 2, 3, 1))
    o, kh, kw, i = w.shape
    return w.reshape(o, kh * kw * i).astype(jnp.bfloat16)


def kernel(x, w1, w2, bn1_gamma, bn1_beta, bn1_mean, bn1_var,
           bn2_gamma, bn2_beta, bn2_mean, bn2_var):
    N, C, H, W = x.shape

    s1, b1 = _fold_bn(bn1_gamma, bn1_beta, bn1_mean, bn1_var)
    s2, b2 = _fold_bn(bn2_gamma, bn2_beta, bn2_mean, bn2_var)
    w1p = _prep_weight(w1, s1)
    w2p = _prep_weight(w2, s2)
    b1c = b1.reshape(C, 1).astype(jnp.float32)
    b2c = b2.reshape(C, 1).astype(jnp.float32)

    x_v = x.reshape(N * C, H, W)

    kernel_fn = functools.partial(_bb_kernel, H=H, W=W)

    out = pl.pallas_call(
        kernel_fn,
        out_shape=jax.ShapeDtypeStruct((N * C, H, W), x.dtype),
        grid=(N,),
        in_specs=[
            pl.BlockSpec((C, H, W), lambda b: (b, 0, 0)),
            pl.BlockSpec((C, 9 * C), lambda b: (0, 0)),
            pl.BlockSpec((C, 1), lambda b: (0, 0)),
            pl.BlockSpec((C, 9 * C), lambda b: (0, 0)),
            pl.BlockSpec((C, 1), lambda b: (0, 0)),
        ],
        out_specs=pl.BlockSpec((C, H, W), lambda b: (b, 0, 0)),
        scratch_shapes=[
            pltpu.VMEM((C, H * W), jnp.float32),
            pltpu.VMEM((9 * C, H * W), jnp.bfloat16),
        ],
        compiler_params=pltpu.CompilerParams(
            dimension_semantics=("parallel",),
            vmem_limit_bytes=48 * 1024 * 1024,
        ),
    )(x_v, w1p, b1c, w2p, b2c)

    return out.reshape(N, C, H, W)


# speedup vs baseline: 2.0614x; 1.7150x over previous
import functools

import jax
import jax.numpy as jnp
from jax import lax
from jax.experimental import pallas as pl
from jax.experimental.pallas import tpu as pltpu


def _bb_kernel(x_ref, w1_ref, b1_ref, w2_ref, b2_ref, out_ref, col_ref,
               *, H, W):
    C, L = x_ref.shape
    HW = H * W

    x = x_ref[...]

    lane = lax.broadcasted_iota(jnp.int32, (1, L), 1)
    colx = lane % W
    rowy = (lane % HW) // W
    col_m = [colx >= 1, None, colx <= W - 2]
    row_m = [rowy >= 1, None, rowy <= H - 2]
    masks = []
    for ky in range(3):
        for kx in range(3):
            m = row_m[ky]
            if col_m[kx] is not None:
                m = col_m[kx] if m is None else jnp.logical_and(m, col_m[kx])
            masks.append(None if m is None else jnp.where(m, 1.0, 0.0))

    def conv3x3(a, w_ref_loc):
        for ky in range(3):
            for kx in range(3):
                tap = ky * 3 + kx
                s = (ky - 1) * W + (kx - 1)
                t = a if s == 0 else pltpu.roll(a, shift=(-s) % L, axis=1)
                m = masks[tap]
                if m is not None:
                    t = t * m
                col_ref[tap * C:(tap + 1) * C, :] = t.astype(jnp.bfloat16)
        return jnp.dot(w_ref_loc[...], col_ref[...],
                       preferred_element_type=jnp.float32)

    y1 = jnp.maximum(conv3x3(x, w1_ref) + b1_ref[...], 0.0)
    y2 = jnp.maximum(conv3x3(y1, w2_ref) + b2_ref[...] + x, 0.0)
    out_ref[...] = y2.astype(out_ref.dtype)


def _fold_bn(gamma, beta, mean, var, eps=1e-5):
    scale = gamma / jnp.sqrt(var + eps)
    bias = beta - mean * scale
    return scale, bias


def _prep_weight(w_oihw, scale):
    w = w_oihw.astype(jnp.float32) * scale[:, None, None, None]
    w = jnp.transpose(w, (0, 2, 3, 1))
    o, kh, kw, i = w.shape
    return w.reshape(o, kh * kw * i).astype(jnp.bfloat16)


def kernel(x, w1, w2, bn1_gamma, bn1_beta, bn1_mean, bn1_var,
           bn2_gamma, bn2_beta, bn2_mean, bn2_var):
    N, C, H, W = x.shape
    HW = H * W
    NB = 2
    steps = N // NB

    s1, b1 = _fold_bn(bn1_gamma, bn1_beta, bn1_mean, bn1_var)
    s2, b2 = _fold_bn(bn2_gamma, bn2_beta, bn2_mean, bn2_var)
    w1p = _prep_weight(w1, s1)
    w2p = _prep_weight(w2, s2)
    b1c = b1.reshape(C, 1).astype(jnp.float32)
    b2c = b2.reshape(C, 1).astype(jnp.float32)

    x_flat = jnp.transpose(x.reshape(N, C, HW), (1, 0, 2)).reshape(C, N * HW)

    kernel_fn = functools.partial(_bb_kernel, H=H, W=W)

    out_flat = pl.pallas_call(
        kernel_fn,
        out_shape=jax.ShapeDtypeStruct((C, N * HW), x.dtype),
        grid=(steps,),
        in_specs=[
            pl.BlockSpec((C, NB * HW), lambda b: (0, b)),
            pl.BlockSpec((C, 9 * C), lambda b: (0, 0)),
            pl.BlockSpec((C, 1), lambda b: (0, 0)),
            pl.BlockSpec((C, 9 * C), lambda b: (0, 0)),
            pl.BlockSpec((C, 1), lambda b: (0, 0)),
        ],
        out_specs=pl.BlockSpec((C, NB * HW), lambda b: (0, b)),
        scratch_shapes=[
            pltpu.VMEM((9 * C, NB * HW), jnp.bfloat16),
        ],
        compiler_params=pltpu.CompilerParams(
            dimension_semantics=("parallel",),
            vmem_limit_bytes=48 * 1024 * 1024,
        ),
    )(x_flat, w1p, b1c, w2p, b2c)

    return out_flat.reshape(C, N, HW).transpose(1, 0, 2).reshape(N, C, H, W)
